# baseline (speedup 1.0000x reference)
import functools

import jax
import jax.numpy as jnp
from jax import lax
from jax.experimental import pallas as pl
from jax.experimental.pallas import tpu as pltpu
from jax.experimental.pallas import tpu_sc as plsc

N = 16384
D = 128
L = 16
NC, NS = 2, 16
NW = NC * NS

N_SC = 0
N_TC = N - N_SC
ROWS_PER_W = max(N_SC // NW, L)
VPR = D // L

_mesh = plsc.VectorSubcoreMesh(core_axis_name="c", subcore_axis_name="s")


def _permute(v, idx):
    dnums = lax.GatherDimensionNumbers(
        offset_dims=(), collapsed_slice_dims=(0,), start_index_map=(0,))
    return lax.gather(v, idx[:, None], dnums, (1,),
                      mode=lax.GatherScatterMode.PROMISE_IN_BOUNDS)


@functools.partial(
    pl.kernel,
    mesh=_mesh,
    out_type=jax.ShapeDtypeStruct((N_SC, D), jnp.float32),
    scratch_types=[
        pltpu.VMEM((ROWS_PER_W, D), jnp.float32),
        pltpu.VMEM((ROWS_PER_W,), jnp.int32),
        pltpu.VMEM((L,), jnp.float32),
        pltpu.VMEM((L,), jnp.float32),
    ],
)
def _affine_sc(z_hbm, obs_hbm, coefs_hbm, offsets_hbm, out_hbm,
               zbuf, obsbuf, cbuf, obuf):
    wid = lax.axis_index("s") * NC + lax.axis_index("c")
    base = wid * ROWS_PER_W

    pltpu.sync_copy(coefs_hbm, cbuf.at[pl.ds(0, 8)])
    pltpu.sync_copy(offsets_hbm, obuf.at[pl.ds(0, 8)])
    pltpu.sync_copy(obs_hbm.at[pl.ds(base, ROWS_PER_W)], obsbuf)
    pltpu.sync_copy(z_hbm.at[pl.ds(base, ROWS_PER_W)], zbuf)

    ctab = cbuf[...]
    otab = obuf[...]

    def group_body(t, carry):
        r0 = t * L
        idx16 = obsbuf[pl.ds(r0, L)]
        c16 = _permute(ctab, idx16)
        o16 = _permute(otab, idx16)
        for k in range(L):
            lane = jnp.full((L,), k, dtype=jnp.int32)
            c = _permute(c16, lane)
            o = _permute(o16, lane)
            for j in range(VPR):
                s = pl.ds(j * L, L)
                zbuf[r0 + k, s] = c * zbuf[r0 + k, s] + o
        return carry

    lax.fori_loop(0, ROWS_PER_W // L, group_body, 0)

    pltpu.sync_copy(zbuf, out_hbm.at[pl.ds(base, ROWS_PER_W)])


TC_BLK = 16384
TC_NB = N_TC // TC_BLK
TC_OFF = N_SC // TC_BLK


def _affine_tc_body(obs_ref, coefs_ref, offsets_ref, z_ref, o_ref):
    ob = obs_ref[0, 0, :]
    c = jnp.zeros((TC_BLK,), jnp.float32)
    o = jnp.zeros((TC_BLK,), jnp.float32)
    for k in range(8):
        sel = ob == k
        c = jnp.where(sel, coefs_ref[k], c)
        o = jnp.where(sel, offsets_ref[k], o)
    o_ref[...] = c[:, None] * z_ref[...] + o[:, None]


def _affine_tc(z, obs, coefs, offsets):
    obs3 = obs.reshape(N // TC_BLK, 1, TC_BLK)
    return pl.pallas_call(
        _affine_tc_body,
        grid=(TC_NB,),
        in_specs=[
            pl.BlockSpec((1, 1, TC_BLK), lambda i: (i + TC_OFF, 0, 0)),
            pl.BlockSpec(memory_space=pltpu.SMEM),
            pl.BlockSpec(memory_space=pltpu.SMEM),
            pl.BlockSpec((TC_BLK, D), lambda i: (i + TC_OFF, 0)),
        ],
        out_specs=pl.BlockSpec((TC_BLK, D), lambda i: (i, 0)),
        out_shape=jax.ShapeDtypeStruct((N_TC, D), jnp.float32),
    )(obs3, coefs, offsets, z)


def kernel(z, obs, coefs, offsets):
    obs32 = obs.astype(jnp.int32)
    if N_SC == 0:
        return _affine_tc(z, obs32, coefs, offsets)
    y_sc = _affine_sc(z, obs32, coefs, offsets)
    y_tc = _affine_tc(z, obs32, coefs, offsets)
    return jnp.concatenate([y_sc, y_tc], axis=0)


# speedup vs baseline: 1.2600x; 1.2600x over previous
import functools

import jax
import jax.numpy as jnp
from jax import lax
from jax.experimental import pallas as pl
from jax.experimental.pallas import tpu as pltpu
from jax.experimental.pallas import tpu_sc as plsc

N = 16384
D = 128
L = 16
NC, NS = 2, 16
NW = NC * NS

N_SC = 0
N_TC = N - N_SC
ROWS_PER_W = max(N_SC // NW, L)
VPR = D // L

_mesh = plsc.VectorSubcoreMesh(core_axis_name="c", subcore_axis_name="s")


def _permute(v, idx):
    dnums = lax.GatherDimensionNumbers(
        offset_dims=(), collapsed_slice_dims=(0,), start_index_map=(0,))
    return lax.gather(v, idx[:, None], dnums, (1,),
                      mode=lax.GatherScatterMode.PROMISE_IN_BOUNDS)


@functools.partial(
    pl.kernel,
    mesh=_mesh,
    out_type=jax.ShapeDtypeStruct((N_SC, D), jnp.float32),
    scratch_types=[
        pltpu.VMEM((ROWS_PER_W, D), jnp.float32),
        pltpu.VMEM((ROWS_PER_W,), jnp.int32),
        pltpu.VMEM((L,), jnp.float32),
        pltpu.VMEM((L,), jnp.float32),
    ],
)
def _affine_sc(z_hbm, obs_hbm, coefs_hbm, offsets_hbm, out_hbm,
               zbuf, obsbuf, cbuf, obuf):
    wid = lax.axis_index("s") * NC + lax.axis_index("c")
    base = wid * ROWS_PER_W

    pltpu.sync_copy(coefs_hbm, cbuf.at[pl.ds(0, 8)])
    pltpu.sync_copy(offsets_hbm, obuf.at[pl.ds(0, 8)])
    pltpu.sync_copy(obs_hbm.at[pl.ds(base, ROWS_PER_W)], obsbuf)
    pltpu.sync_copy(z_hbm.at[pl.ds(base, ROWS_PER_W)], zbuf)

    ctab = cbuf[...]
    otab = obuf[...]

    def group_body(t, carry):
        r0 = t * L
        idx16 = obsbuf[pl.ds(r0, L)]
        c16 = _permute(ctab, idx16)
        o16 = _permute(otab, idx16)
        for k in range(L):
            lane = jnp.full((L,), k, dtype=jnp.int32)
            c = _permute(c16, lane)
            o = _permute(o16, lane)
            for j in range(VPR):
                s = pl.ds(j * L, L)
                zbuf[r0 + k, s] = c * zbuf[r0 + k, s] + o
        return carry

    lax.fori_loop(0, ROWS_PER_W // L, group_body, 0)

    pltpu.sync_copy(zbuf, out_hbm.at[pl.ds(base, ROWS_PER_W)])


TC_BLK = 8192
TC_NB = N_TC // TC_BLK
TC_OFF = N_SC // TC_BLK


def _affine_tc_body(obs_ref, coefs_ref, offsets_ref, z_ref, o_ref):
    ob = obs_ref[0, 0, :]
    c = jnp.zeros((TC_BLK,), jnp.float32)
    o = jnp.zeros((TC_BLK,), jnp.float32)
    for k in range(8):
        sel = ob == k
        c = jnp.where(sel, coefs_ref[k], c)
        o = jnp.where(sel, offsets_ref[k], o)
    o_ref[...] = c[:, None] * z_ref[...] + o[:, None]


def _affine_tc(z, obs, coefs, offsets):
    obs3 = obs.reshape(N // TC_BLK, 1, TC_BLK)
    return pl.pallas_call(
        _affine_tc_body,
        grid=(TC_NB,),
        in_specs=[
            pl.BlockSpec((1, 1, TC_BLK), lambda i: (i + TC_OFF, 0, 0)),
            pl.BlockSpec(memory_space=pltpu.SMEM),
            pl.BlockSpec(memory_space=pltpu.SMEM),
            pl.BlockSpec((TC_BLK, D), lambda i: (i + TC_OFF, 0)),
        ],
        out_specs=pl.BlockSpec((TC_BLK, D), lambda i: (i, 0)),
        out_shape=jax.ShapeDtypeStruct((N_TC, D), jnp.float32),
    )(obs3, coefs, offsets, z)


def kernel(z, obs, coefs, offsets):
    obs32 = obs.astype(jnp.int32)
    if N_SC == 0:
        return _affine_tc(z, obs32, coefs, offsets)
    y_sc = _affine_sc(z, obs32, coefs, offsets)
    y_tc = _affine_tc(z, obs32, coefs, offsets)
    return jnp.concatenate([y_sc, y_tc], axis=0)
